# speedup vs baseline: 2.1525x; 2.1525x over previous
import functools

import jax
import jax.numpy as jnp
from jax import lax
from jax.experimental import pallas as pl
from jax.experimental.pallas import tpu as pltpu
from jax.experimental.pallas import tpu_sc as plsc

D = 64
ROW = 2 * D
NC, NS = 2, 16
NW = NC * NS
CHUNK = 128


def _relu(x):
    return jnp.maximum(x, 0.0)


@functools.lru_cache(maxsize=None)
def _build(batch, num_classes):
    pw = batch // NW
    nchunk = pw // CHUNK

    mesh = plsc.VectorSubcoreMesh(core_axis_name="c", subcore_axis_name="s")

    @functools.partial(
        pl.kernel,
        mesh=mesh,
        out_type=jax.ShapeDtypeStruct((NW, 16), jnp.float32),
        scratch_types=[
            pltpu.VMEM((2, nchunk, CHUNK), jnp.int32),
            pltpu.VMEM((3, nchunk, CHUNK), jnp.int32),
            pltpu.VMEM((CHUNK, ROW), jnp.float32),
            pltpu.VMEM((CHUNK, ROW), jnp.float32),
            pltpu.VMEM((CHUNK, ROW), jnp.float32),
            pltpu.VMEM((16,), jnp.float32),
            pltpu.SemaphoreType.DMA,
        ],
    )
    def k(nf1_hbm, nf2_hbm, emb_hbm, out_hbm, idx1, idx2, rc, rd, re, accv,
          sem):
        wid = lax.axis_index("s") * NC + lax.axis_index("c")

        pltpu.sync_copy(nf1_hbm.at[0, wid], idx1.at[0])
        pltpu.sync_copy(nf1_hbm.at[1, wid], idx1.at[1])
        pltpu.sync_copy(nf2_hbm.at[0, wid], idx2.at[0])
        pltpu.sync_copy(nf2_hbm.at[1, wid], idx2.at[1])
        pltpu.sync_copy(nf2_hbm.at[2, wid], idx2.at[2])

        acc0 = jnp.zeros((16,), jnp.float32)

        def nf1_chunk(g, acc):
            cp_c = pltpu.async_copy(emb_hbm.at[idx1.at[0, g]], rc, sem)
            cp_d = pltpu.async_copy(emb_hbm.at[idx1.at[1, g]], rd, sem)
            cp_c.wait()
            cp_d.wait()

            def body(i, a):
                for j in range(4):
                    cC = rc[i, pl.ds(16 * j, 16)]
                    cO = rc[i, pl.ds(D + 16 * j, 16)]
                    dC = rd[i, pl.ds(16 * j, 16)]
                    dO = rd[i, pl.ds(D + 16 * j, 16)]
                    a = a + (_relu(dC - cC) + _relu(cO - dO)
                             + _relu(cC - cO) + _relu(dC - dO))
                return a

            return lax.fori_loop(0, CHUNK, body, acc)

        acc1 = lax.fori_loop(0, nchunk, nf1_chunk, acc0)

        def nf2_chunk(g, acc):
            cp_c = pltpu.async_copy(emb_hbm.at[idx2.at[0, g]], rc, sem)
            cp_d = pltpu.async_copy(emb_hbm.at[idx2.at[1, g]], rd, sem)
            cp_e = pltpu.async_copy(emb_hbm.at[idx2.at[2, g]], re, sem)
            cp_c.wait()
            cp_d.wait()
            cp_e.wait()

            def body(i, a):
                for j in range(4):
                    cC = rc[i, pl.ds(16 * j, 16)]
                    cO = rc[i, pl.ds(D + 16 * j, 16)]
                    dC = rd[i, pl.ds(16 * j, 16)]
                    dO = rd[i, pl.ds(D + 16 * j, 16)]
                    eC = re[i, pl.ds(16 * j, 16)]
                    eO = re[i, pl.ds(D + 16 * j, 16)]
                    start_all = jnp.maximum(cC, dC)
                    end_all = jnp.minimum(cO, dO)
                    a = a + (_relu(eC - start_all) + _relu(end_all - eO)
                             + _relu(cC - cO) + _relu(dC - dO)
                             + _relu(eC - eO))
                return a

            return lax.fori_loop(0, CHUNK, body, acc)

        acc2 = lax.fori_loop(0, nchunk, nf2_chunk, acc1)

        accv[...] = acc2
        pltpu.sync_copy(accv, out_hbm.at[wid])

    return k


def kernel(nf1, nf2, classEmb):
    batch = nf1.shape[0]
    num_classes = classEmb.shape[0]
    del num_classes
    pw = batch // NW
    nchunk = pw // CHUNK
    nf1_r = nf1.T.reshape(2, NW, nchunk, CHUNK)
    nf2_r = nf2.T.reshape(3, NW, nchunk, CHUNK)
    out = _build(batch, nchunk)(nf1_r, nf2_r, classEmb)
    return jnp.sum(out) / jnp.float32(batch)


# speedup vs baseline: 2.5912x; 1.2038x over previous
import functools

import jax
import jax.numpy as jnp
from jax import lax
from jax.experimental import pallas as pl
from jax.experimental.pallas import tpu as pltpu
from jax.experimental.pallas import tpu_sc as plsc

D = 64
ROW = 2 * D
NC, NS = 2, 16
NW = NC * NS
CHUNK = 128


def _relu(x):
    return jnp.maximum(x, 0.0)


@functools.lru_cache(maxsize=None)
def _build(batch):
    pw = batch // NW
    nchunk = pw // CHUNK

    mesh = plsc.VectorSubcoreMesh(core_axis_name="c", subcore_axis_name="s")

    @functools.partial(
        pl.kernel,
        mesh=mesh,
        out_type=jax.ShapeDtypeStruct((NW, 16), jnp.float32),
        scratch_types=[
            pltpu.VMEM((2, nchunk, CHUNK), jnp.int32),
            pltpu.VMEM((3, nchunk, CHUNK), jnp.int32),
            pltpu.VMEM((2, CHUNK, ROW), jnp.float32),
            pltpu.VMEM((2, CHUNK, ROW), jnp.float32),
            pltpu.VMEM((2, CHUNK, ROW), jnp.float32),
            pltpu.VMEM((16,), jnp.float32),
            pltpu.SemaphoreType.DMA,
        ],
    )
    def k(nf1_hbm, nf2_hbm, emb_hbm, out_hbm, idx1, idx2, rc, rd, re, accv,
          sem):
        wid = lax.axis_index("s") * NC + lax.axis_index("c")

        pltpu.sync_copy(nf1_hbm.at[0, wid], idx1.at[0])
        pltpu.sync_copy(nf1_hbm.at[1, wid], idx1.at[1])
        pltpu.sync_copy(nf2_hbm.at[0, wid], idx2.at[0])
        pltpu.sync_copy(nf2_hbm.at[1, wid], idx2.at[1])
        pltpu.sync_copy(nf2_hbm.at[2, wid], idx2.at[2])

        tasks = [(1, g) for g in range(nchunk)] + [(2, g) for g in range(nchunk)]

        def issue(t):
            ph, g = tasks[t]
            b = t & 1
            if ph == 1:
                return [
                    pltpu.async_copy(emb_hbm.at[idx1.at[0, g]], rc.at[b], sem),
                    pltpu.async_copy(emb_hbm.at[idx1.at[1, g]], rd.at[b], sem),
                ]
            return [
                pltpu.async_copy(emb_hbm.at[idx2.at[0, g]], rc.at[b], sem),
                pltpu.async_copy(emb_hbm.at[idx2.at[1, g]], rd.at[b], sem),
                pltpu.async_copy(emb_hbm.at[idx2.at[2, g]], re.at[b], sem),
            ]

        def compute_nf1(b, accs):
            def body(i, a):
                out = list(a)
                for j in range(4):
                    cC = rc[b, i, pl.ds(16 * j, 16)]
                    cO = rc[b, i, pl.ds(D + 16 * j, 16)]
                    dC = rd[b, i, pl.ds(16 * j, 16)]
                    dO = rd[b, i, pl.ds(D + 16 * j, 16)]
                    out[j] = out[j] + (_relu(dC - cC) + _relu(cO - dO)
                                       + _relu(cC - cO) + _relu(dC - dO))
                return tuple(out)

            return lax.fori_loop(0, CHUNK, body, accs)

        def compute_nf2(b, accs):
            def body(i, a):
                out = list(a)
                for j in range(4):
                    cC = rc[b, i, pl.ds(16 * j, 16)]
                    cO = rc[b, i, pl.ds(D + 16 * j, 16)]
                    dC = rd[b, i, pl.ds(16 * j, 16)]
                    dO = rd[b, i, pl.ds(D + 16 * j, 16)]
                    eC = re[b, i, pl.ds(16 * j, 16)]
                    eO = re[b, i, pl.ds(D + 16 * j, 16)]
                    start_all = jnp.maximum(cC, dC)
                    end_all = jnp.minimum(cO, dO)
                    out[j] = out[j] + (_relu(eC - start_all)
                                       + _relu(end_all - eO)
                                       + _relu(cC - cO) + _relu(dC - dO)
                                       + _relu(eC - eO))
                return tuple(out)

            return lax.fori_loop(0, CHUNK, body, accs)

        zero = jnp.zeros((16,), jnp.float32)
        accs = (zero, zero, zero, zero)

        cps = issue(0)
        for t in range(len(tasks)):
            for c in cps:
                c.wait()
            nxt = issue(t + 1) if t + 1 < len(tasks) else []
            ph, _ = tasks[t]
            b = t & 1
            if ph == 1:
                accs = compute_nf1(b, accs)
            else:
                accs = compute_nf2(b, accs)
            cps = nxt

        accv[...] = accs[0] + accs[1] + accs[2] + accs[3]
        pltpu.sync_copy(accv, out_hbm.at[wid])

    return k


def kernel(nf1, nf2, classEmb):
    batch = nf1.shape[0]
    pw = batch // NW
    nchunk = pw // CHUNK
    nf1_r = nf1.T.reshape(2, NW, nchunk, CHUNK)
    nf2_r = nf2.T.reshape(3, NW, nchunk, CHUNK)
    out = _build(batch)(nf1_r, nf2_r, classEmb)
    return jnp.sum(out) / jnp.float32(batch)
